# Initial kernel scaffold; baseline (speedup 1.0000x reference)
import jax
import jax.numpy as jnp
from jax.experimental import pallas as pl


def kernel(x, edge_index, batch, W_self, b_self, W_neigh, W_fp):
    raise NotImplementedError("write your pallas kernel here")


# speedup vs baseline: 18.9504x; 18.9504x over previous
import functools

import jax
import jax.numpy as jnp
from jax import lax
from jax.experimental import pallas as pl
from jax.experimental.pallas import tpu as pltpu
from jax.experimental.pallas import tpu_sc as plsc

NUM_GRAPHS = 64
TILE = 256
CHUNK = 128
LN = 16
KEY_SHIFT = 16384


def _sc_mesh():
    return plsc.VectorSubcoreMesh(core_axis_name="c", subcore_axis_name="s")


def _make_deg_kernel(n_pad, e_pad, nc, ns):
    nw = nc * ns
    ep_w = e_pad // nw
    nchunks = ep_w // CHUNK
    rows_t = n_pad // ns

    @functools.partial(
        pl.kernel,
        out_type=jax.ShapeDtypeStruct((nc, n_pad), jnp.int32),
        mesh=_sc_mesh(),
        scratch_types=[
            pltpu.VMEM((CHUNK,), jnp.int32),
            pltpu.VMEM((CHUNK,), jnp.int32),
            pltpu.VMEM_SHARED((n_pad,), jnp.int32),
        ],
    )
    def deg_kernel(row_hbm, zero_hbm, out_hbm, idx_v, ones_v, hist_sh):
        cid = lax.axis_index("c")
        sid = lax.axis_index("s")
        wid = sid * nc + cid
        r0 = sid * rows_t
        pltpu.sync_copy(zero_hbm.at[pl.ds(r0, rows_t)],
                        hist_sh.at[pl.ds(r0, rows_t)])
        for k in range(CHUNK // LN):
            ones_v[pl.ds(k * LN, LN)] = jnp.ones((LN,), jnp.int32)
        plsc.subcore_barrier()

        def chunk(c, _):
            base = wid * ep_w + c * CHUNK
            pltpu.sync_copy(row_hbm.at[pl.ds(base, CHUNK)], idx_v)
            pltpu.sync_copy(ones_v, hist_sh.at[idx_v], add=True)
            return _

        lax.fori_loop(0, nchunks, chunk, 0)
        plsc.subcore_barrier()
        pltpu.sync_copy(hist_sh.at[pl.ds(r0, rows_t)],
                        out_hbm.at[cid].at[pl.ds(r0, rows_t)])

    return deg_kernel


def _make_dval_kernel(e_pad, nc, ns):
    nw = nc * ns
    ep_w = e_pad // nw
    nchunks = ep_w // CHUNK

    @functools.partial(
        pl.kernel,
        out_type=jax.ShapeDtypeStruct((e_pad,), jnp.int32),
        mesh=_sc_mesh(),
        scratch_types=[
            pltpu.VMEM((CHUNK,), jnp.int32),
            pltpu.VMEM((CHUNK,), jnp.int32),
            pltpu.SemaphoreType.DMA,
        ],
    )
    def dval_kernel(row_hbm, deg_hbm, out_hbm, idx_v, dv_v, sem):
        cid = lax.axis_index("c")
        sid = lax.axis_index("s")
        wid = sid * nc + cid

        def chunk(c, _):
            base = wid * ep_w + c * CHUNK
            pltpu.sync_copy(row_hbm.at[pl.ds(base, CHUNK)], idx_v)
            pltpu.async_copy(deg_hbm.at[idx_v], dv_v, sem).wait()
            pltpu.sync_copy(dv_v, out_hbm.at[pl.ds(base, CHUNK)])
            return _

        lax.fori_loop(0, nchunks, chunk, 0)

    return dval_kernel


def _make_scatter_kernel(n_pad, e_alloc, h, dp1, nc, ns):
    nw = nc * ns
    half = n_pad // nc
    rows_t = n_pad // nw
    acc_rows = half + CHUNK
    zrows = 64

    @functools.partial(
        pl.kernel,
        out_type=jax.ShapeDtypeStruct((dp1 - 1, n_pad, h), jnp.float32),
        mesh=_sc_mesh(),
        scratch_types=[
            pltpu.VMEM((48,), jnp.int32),
            pltpu.VMEM((48,), jnp.int32),
            pltpu.VMEM((CHUNK,), jnp.int32),
            pltpu.VMEM((CHUNK,), jnp.int32),
            pltpu.VMEM((CHUNK,), jnp.int32),
            pltpu.VMEM((zrows, h), jnp.float32),
            pltpu.VMEM((CHUNK, h), jnp.float32),
            pltpu.VMEM_SHARED((acc_rows, h), jnp.float32),
            pltpu.SemaphoreType.DMA,
        ],
    )
    def scatter_kernel(x_hbm, row_hbm, col_hbm, s2_hbm, e2_hbm, zero_hbm,
                       out_hbm, s_v, e_v, row_v, col_v, cloc_v, zbuf, gbuf,
                       acc_sh, sem):
        cid = lax.axis_index("c")
        sid = lax.axis_index("s")
        w2 = cid * ns + sid
        dst_lo = w2 * rows_t
        loc0 = sid * rows_t
        pltpu.sync_copy(s2_hbm.at[w2], s_v)
        pltpu.sync_copy(e2_hbm.at[w2], e_v)
        pltpu.sync_copy(zero_hbm.at[pl.ds(0, zrows)], zbuf)
        col_base = cid * half

        s_vecs = [s_v[pl.ds(0, LN)], s_v[pl.ds(LN, LN)]]
        e_vecs = [e_v[pl.ds(0, LN)], e_v[pl.ds(LN, LN)]]
        for dm1 in range(dp1 - 1):
            s = s_vecs[dm1 // LN][dm1 % LN]
            e = e_vecs[dm1 // LN][dm1 % LN]
            s8 = (s // 8) * 8
            niter = lax.div(e - s8 + CHUNK - 1, CHUNK)
            for j in range(rows_t // zrows):
                pltpu.sync_copy(zbuf, acc_sh.at[pl.ds(loc0 + j * zrows, zrows)])

            def chunk(i, _):
                cstart = s8 + i * CHUNK
                pltpu.sync_copy(row_hbm.at[pl.ds(cstart, CHUNK)], row_v)
                pltpu.sync_copy(col_hbm.at[pl.ds(cstart, CHUNK)], col_v)
                for k in range(CHUNK // LN):
                    jk = cstart + k * LN + lax.iota(jnp.int32, LN)
                    cv = col_v[pl.ds(k * LN, LN)]
                    valid = jnp.logical_and(jk >= s, jk < e)
                    cloc = jnp.where(valid, cv - col_base, half)
                    cloc_v[pl.ds(k * LN, LN)] = cloc
                pltpu.async_copy(x_hbm.at[row_v], gbuf, sem).wait()
                pltpu.sync_copy(gbuf, acc_sh.at[cloc_v], add=True)
                return _

            lax.fori_loop(0, niter, chunk, 0)
            pltpu.sync_copy(acc_sh.at[pl.ds(loc0, rows_t)],
                            out_hbm.at[dm1].at[pl.ds(dst_lo, rows_t)])

    return scatter_kernel


def _tc_body(dp1, batch_ref, x_ref, msgs_ref, wst_ref, b_ref, wnt_ref,
             wfpt_ref, fpin_ref, h_ref, fp_ref):
    i = pl.program_id(0)
    xb = x_ref[...]
    hs = jnp.dot(xb, wst_ref[...], preferred_element_type=jnp.float32) + b_ref[...]
    neigh = jnp.dot(msgs_ref[0], wnt_ref[0], preferred_element_type=jnp.float32)
    for d in range(1, dp1 - 1):
        neigh = neigh + jnp.dot(msgs_ref[d], wnt_ref[d],
                                preferred_element_type=jnp.float32)
    hh = jnp.tanh(hs + neigh)
    h_ref[...] = hh
    logits = jnp.dot(hh, wfpt_ref[...], preferred_element_type=jnp.float32)
    mx = jnp.max(logits, axis=1, keepdims=True)
    p = jnp.exp(logits - mx)
    contrib = p / jnp.sum(p, axis=1, keepdims=True)
    b = batch_ref[0, 0, :]
    gids = lax.broadcasted_iota(jnp.int32, (NUM_GRAPHS, TILE), 0)
    oh = (gids == b[None, :]).astype(jnp.float32)
    fpt = jnp.dot(oh, contrib, preferred_element_type=jnp.float32)

    @pl.when(i == 0)
    def _():
        fp_ref[...] = fpin_ref[...] + fpt

    @pl.when(i > 0)
    def _():
        fp_ref[...] = fp_ref[...] + fpt


def kernel(x, edge_index, batch, W_self, b_self, W_neigh, W_fp):
    n, in_dim = x.shape
    num_layers, hid, _ = W_self.shape
    dp1 = W_neigh.shape[1]
    fp_dim = W_fp.shape[1]
    e = edge_index.shape[1]

    info = plsc.get_sparse_core_info()
    nc, ns = info.num_cores, info.num_subcores
    nw = nc * ns

    n_pad = ((n + TILE - 1) // TILE) * TILE
    ntiles = n_pad // TILE
    e_pad = ((e + nw * CHUNK - 1) // (nw * CHUNK)) * (nw * CHUNK)
    e_alloc = e_pad + CHUNK
    rows_t = n_pad // nw

    row = edge_index[0]
    col = edge_index[1]
    row_p = jnp.concatenate([row, jnp.full((e_pad - e,), n, jnp.int32)])
    col_p = jnp.concatenate([col, jnp.full((e_pad - e,), n, jnp.int32)])
    x_p = jnp.zeros((n_pad, in_dim), jnp.float32).at[:n].set(x)
    batch_p = jnp.concatenate(
        [batch, jnp.full((n_pad - n,), NUM_GRAPHS, jnp.int32)]
    ).reshape(ntiles, 1, TILE)

    wst = W_self.transpose(0, 2, 1)
    wnt = W_neigh.transpose(0, 1, 3, 2)[:, 1:]
    wfpt = W_fp.transpose(0, 2, 1)
    b2 = b_self.reshape(num_layers, 1, hid)

    zero_i = jnp.zeros((n_pad,), jnp.int32)
    zero_f = jnp.zeros((n_pad, hid), jnp.float32)

    deg_parts = _make_deg_kernel(n_pad, e_pad, nc, ns)(row_p, zero_i)
    deg_full = deg_parts[0] + deg_parts[1]

    dvals = _make_dval_kernel(e_pad, nc, ns)(row_p, deg_full)

    dkey = jnp.minimum(dvals, dp1)
    key = dkey * KEY_SHIFT + col_p
    perm = jnp.argsort(key, stable=True)
    key_s = key[perm]
    row_s = jnp.concatenate([row_p[perm], jnp.full((CHUNK,), n, jnp.int32)])
    col_s = jnp.concatenate([col_p[perm], jnp.full((CHUNK,), n, jnp.int32)])
    probes = (jnp.arange(1, dp1)[:, None] * KEY_SHIFT
              + jnp.arange(nw + 1)[None, :] * rows_t)
    bnd = jnp.searchsorted(key_s, probes.reshape(-1), side="left")
    bnd = bnd.reshape(dp1 - 1, nw + 1).astype(jnp.int32)
    s2 = jnp.zeros((nw, 48), jnp.int32).at[:, : dp1 - 1].set(bnd[:, :nw].T)
    e2 = jnp.zeros((nw, 48), jnp.int32).at[:, : dp1 - 1].set(bnd[:, 1:].T)

    scatter_kernel = _make_scatter_kernel(n_pad, e_alloc, hid, dp1, nc, ns)

    tc_f = pl.pallas_call(
        functools.partial(_tc_body, dp1),
        grid=(ntiles,),
        in_specs=[
            pl.BlockSpec((1, 1, TILE), lambda i: (i, 0, 0)),
            pl.BlockSpec((TILE, in_dim), lambda i: (i, 0)),
            pl.BlockSpec((dp1 - 1, TILE, hid), lambda i: (0, i, 0)),
            pl.BlockSpec((in_dim, hid), lambda i: (0, 0)),
            pl.BlockSpec((1, hid), lambda i: (0, 0)),
            pl.BlockSpec((dp1 - 1, in_dim, hid), lambda i: (0, 0, 0)),
            pl.BlockSpec((hid, fp_dim), lambda i: (0, 0)),
            pl.BlockSpec((NUM_GRAPHS, fp_dim), lambda i: (0, 0)),
        ],
        out_specs=[
            pl.BlockSpec((TILE, hid), lambda i: (i, 0)),
            pl.BlockSpec((NUM_GRAPHS, fp_dim), lambda i: (0, 0)),
        ],
        out_shape=[
            jax.ShapeDtypeStruct((n_pad, hid), jnp.float32),
            jax.ShapeDtypeStruct((NUM_GRAPHS, fp_dim), jnp.float32),
        ],
    )

    fp = jnp.zeros((NUM_GRAPHS, fp_dim), jnp.float32)
    xx = x_p
    for layer in range(num_layers):
        msgs = scatter_kernel(xx, row_s, col_s, s2, e2, zero_f)
        xx, fp = tc_f(batch_p, xx, msgs, wst[layer], b2[layer],
                      wnt[layer], wfpt[layer], fp)
    return fp


# speedup vs baseline: 25.4235x; 1.3416x over previous
import functools

import jax
import jax.numpy as jnp
from jax import lax
from jax.experimental import pallas as pl
from jax.experimental.pallas import tpu as pltpu
from jax.experimental.pallas import tpu_sc as plsc

NUM_GRAPHS = 64
TILE = 256
CHUNK = 128
LN = 16
KEY_SHIFT = 16384


def _sc_mesh():
    return plsc.VectorSubcoreMesh(core_axis_name="c", subcore_axis_name="s")


def _make_deg_kernel(n_pad, e_pad, nc, ns):
    nw = nc * ns
    ep_w = e_pad // nw
    nchunks = ep_w // CHUNK
    rows_t = n_pad // ns

    @functools.partial(
        pl.kernel,
        out_type=jax.ShapeDtypeStruct((nc, n_pad), jnp.int32),
        mesh=_sc_mesh(),
        scratch_types=[
            pltpu.VMEM((CHUNK,), jnp.int32),
            pltpu.VMEM((CHUNK,), jnp.int32),
            pltpu.VMEM_SHARED((n_pad,), jnp.int32),
        ],
    )
    def deg_kernel(row_hbm, zero_hbm, out_hbm, idx_v, ones_v, hist_sh):
        cid = lax.axis_index("c")
        sid = lax.axis_index("s")
        wid = sid * nc + cid
        r0 = sid * rows_t
        pltpu.sync_copy(zero_hbm.at[pl.ds(r0, rows_t)],
                        hist_sh.at[pl.ds(r0, rows_t)])
        for k in range(CHUNK // LN):
            ones_v[pl.ds(k * LN, LN)] = jnp.ones((LN,), jnp.int32)
        plsc.subcore_barrier()

        def chunk(c, _):
            base = wid * ep_w + c * CHUNK
            pltpu.sync_copy(row_hbm.at[pl.ds(base, CHUNK)], idx_v)
            pltpu.sync_copy(ones_v, hist_sh.at[idx_v], add=True)
            return _

        lax.fori_loop(0, nchunks, chunk, 0)
        plsc.subcore_barrier()
        pltpu.sync_copy(hist_sh.at[pl.ds(r0, rows_t)],
                        out_hbm.at[cid].at[pl.ds(r0, rows_t)])

    return deg_kernel


def _make_dval_kernel(e_pad, nc, ns):
    nw = nc * ns
    ep_w = e_pad // nw
    nchunks = ep_w // CHUNK

    @functools.partial(
        pl.kernel,
        out_type=jax.ShapeDtypeStruct((e_pad,), jnp.int32),
        mesh=_sc_mesh(),
        scratch_types=[
            pltpu.VMEM((CHUNK,), jnp.int32),
            pltpu.VMEM((CHUNK,), jnp.int32),
            pltpu.SemaphoreType.DMA,
        ],
    )
    def dval_kernel(row_hbm, deg_hbm, out_hbm, idx_v, dv_v, sem):
        cid = lax.axis_index("c")
        sid = lax.axis_index("s")
        wid = sid * nc + cid

        def chunk(c, _):
            base = wid * ep_w + c * CHUNK
            pltpu.sync_copy(row_hbm.at[pl.ds(base, CHUNK)], idx_v)
            pltpu.async_copy(deg_hbm.at[idx_v], dv_v, sem).wait()
            pltpu.sync_copy(dv_v, out_hbm.at[pl.ds(base, CHUNK)])
            return _

        lax.fori_loop(0, nchunks, chunk, 0)

    return dval_kernel


def _make_scatter_kernel(n_pad, e_alloc, h, dp1, nc, ns):
    nw = nc * ns
    half = n_pad // nc
    rows_t = n_pad // nw
    acc_rows = half + CHUNK

    @functools.partial(
        pl.kernel,
        out_type=jax.ShapeDtypeStruct((dp1 - 1, n_pad, h), jnp.float32),
        mesh=_sc_mesh(),
        scratch_types=[
            pltpu.VMEM((48,), jnp.int32),
            pltpu.VMEM((48,), jnp.int32),
            pltpu.VMEM((CHUNK,), jnp.int32),
            pltpu.VMEM((CHUNK,), jnp.int32),
            pltpu.VMEM((CHUNK,), jnp.int32),
            pltpu.VMEM((rows_t, h), jnp.float32),
            pltpu.VMEM((CHUNK, h), jnp.float32),
            pltpu.VMEM_SHARED((acc_rows, h), jnp.float32),
            pltpu.SemaphoreType.DMA,
        ],
    )
    def scatter_kernel(x_hbm, packed_hbm, s2_hbm, e2_hbm, zero_hbm,
                       out_hbm, s_v, e_v, pk_v, row_v, cloc_v, zbuf, gbuf,
                       acc_sh, sem):
        cid = lax.axis_index("c")
        sid = lax.axis_index("s")
        w2 = cid * ns + sid
        dst_lo = w2 * rows_t
        loc0 = sid * rows_t
        pltpu.sync_copy(s2_hbm.at[w2], s_v)
        pltpu.sync_copy(e2_hbm.at[w2], e_v)
        pltpu.sync_copy(zero_hbm.at[pl.ds(0, rows_t)], zbuf)
        col_base = cid * half

        s_vecs = [s_v[pl.ds(0, LN)], s_v[pl.ds(LN, LN)]]
        e_vecs = [e_v[pl.ds(0, LN)], e_v[pl.ds(LN, LN)]]
        for dm1 in range(dp1 - 1):
            s = s_vecs[dm1 // LN][dm1 % LN]
            e = e_vecs[dm1 // LN][dm1 % LN]
            s8 = (s // 8) * 8
            niter = lax.div(e - s8 + CHUNK - 1, CHUNK)
            pltpu.sync_copy(zbuf, acc_sh.at[pl.ds(loc0, rows_t)])

            def chunk(i, _):
                cstart = s8 + i * CHUNK
                pltpu.sync_copy(packed_hbm.at[pl.ds(cstart, CHUNK)], pk_v)
                for k in range(CHUNK // LN):
                    jk = cstart + k * LN + lax.iota(jnp.int32, LN)
                    pv = pk_v[pl.ds(k * LN, LN)]
                    rv = lax.shift_right_logical(pv, 14)
                    cv = jnp.bitwise_and(pv, KEY_SHIFT - 1)
                    valid = jnp.logical_and(jk >= s, jk < e)
                    cloc = jnp.where(valid, cv - col_base, half)
                    row_v[pl.ds(k * LN, LN)] = rv
                    cloc_v[pl.ds(k * LN, LN)] = cloc
                pltpu.async_copy(x_hbm.at[row_v], gbuf, sem).wait()
                pltpu.sync_copy(gbuf, acc_sh.at[cloc_v], add=True)
                return _

            lax.fori_loop(0, niter, chunk, 0)
            pltpu.sync_copy(acc_sh.at[pl.ds(loc0, rows_t)],
                            out_hbm.at[dm1].at[pl.ds(dst_lo, rows_t)])

    return scatter_kernel


def _tc_body(dp1, batch_ref, x_ref, msgs_ref, wst_ref, b_ref, wnt_ref,
             wfpt_ref, fpin_ref, h_ref, fp_ref):
    i = pl.program_id(0)
    xb = x_ref[...]
    hs = jnp.dot(xb, wst_ref[...], preferred_element_type=jnp.float32) + b_ref[...]
    neigh = jnp.dot(msgs_ref[0], wnt_ref[0], preferred_element_type=jnp.float32)
    for d in range(1, dp1 - 1):
        neigh = neigh + jnp.dot(msgs_ref[d], wnt_ref[d],
                                preferred_element_type=jnp.float32)
    hh = jnp.tanh(hs + neigh)
    h_ref[...] = hh
    logits = jnp.dot(hh, wfpt_ref[...], preferred_element_type=jnp.float32)
    mx = jnp.max(logits, axis=1, keepdims=True)
    p = jnp.exp(logits - mx)
    contrib = p / jnp.sum(p, axis=1, keepdims=True)
    b = batch_ref[0, 0, :]
    gids = lax.broadcasted_iota(jnp.int32, (NUM_GRAPHS, TILE), 0)
    oh = (gids == b[None, :]).astype(jnp.float32)
    fpt = jnp.dot(oh, contrib, preferred_element_type=jnp.float32)

    @pl.when(i == 0)
    def _():
        fp_ref[...] = fpin_ref[...] + fpt

    @pl.when(i > 0)
    def _():
        fp_ref[...] = fp_ref[...] + fpt


def kernel(x, edge_index, batch, W_self, b_self, W_neigh, W_fp):
    n, in_dim = x.shape
    num_layers, hid, _ = W_self.shape
    dp1 = W_neigh.shape[1]
    fp_dim = W_fp.shape[1]
    e = edge_index.shape[1]

    info = plsc.get_sparse_core_info()
    nc, ns = info.num_cores, info.num_subcores
    nw = nc * ns

    n_pad = ((n + TILE - 1) // TILE) * TILE
    ntiles = n_pad // TILE
    e_pad = ((e + nw * CHUNK - 1) // (nw * CHUNK)) * (nw * CHUNK)
    e_alloc = e_pad + CHUNK
    rows_t = n_pad // nw

    row = edge_index[0]
    col = edge_index[1]
    row_p = jnp.concatenate([row, jnp.full((e_pad - e,), n, jnp.int32)])
    col_p = jnp.concatenate([col, jnp.full((e_pad - e,), n, jnp.int32)])
    x_p = jnp.zeros((n_pad, in_dim), jnp.float32).at[:n].set(x)
    batch_p = jnp.concatenate(
        [batch, jnp.full((n_pad - n,), NUM_GRAPHS, jnp.int32)]
    ).reshape(ntiles, 1, TILE)

    wst = W_self.transpose(0, 2, 1)
    wnt = W_neigh.transpose(0, 1, 3, 2)[:, 1:]
    wfpt = W_fp.transpose(0, 2, 1)
    b2 = b_self.reshape(num_layers, 1, hid)

    zero_i = jnp.zeros((n_pad,), jnp.int32)
    zero_f = jnp.zeros((n_pad, hid), jnp.float32)

    deg_parts = _make_deg_kernel(n_pad, e_pad, nc, ns)(row_p, zero_i)
    deg_full = deg_parts[0] + deg_parts[1]

    dvals = _make_dval_kernel(e_pad, nc, ns)(row_p, deg_full)

    dkey = jnp.minimum(dvals, dp1)
    key = dkey * KEY_SHIFT + col_p
    packed = row_p * KEY_SHIFT + col_p
    key_s, packed_s = lax.sort([key, packed], dimension=0, is_stable=True,
                               num_keys=1)
    packed_s = jnp.concatenate(
        [packed_s, jnp.full((CHUNK,), n * KEY_SHIFT + n, jnp.int32)])
    probes = (jnp.arange(1, dp1)[:, None] * KEY_SHIFT
              + jnp.arange(nw + 1)[None, :] * rows_t)
    bnd = jnp.searchsorted(key_s, probes.reshape(-1), side="left")
    bnd = bnd.reshape(dp1 - 1, nw + 1).astype(jnp.int32)
    s2 = jnp.zeros((nw, 48), jnp.int32).at[:, : dp1 - 1].set(bnd[:, :nw].T)
    e2 = jnp.zeros((nw, 48), jnp.int32).at[:, : dp1 - 1].set(bnd[:, 1:].T)

    scatter_kernel = _make_scatter_kernel(n_pad, e_alloc, hid, dp1, nc, ns)

    tc_f = pl.pallas_call(
        functools.partial(_tc_body, dp1),
        grid=(ntiles,),
        in_specs=[
            pl.BlockSpec((1, 1, TILE), lambda i: (i, 0, 0)),
            pl.BlockSpec((TILE, in_dim), lambda i: (i, 0)),
            pl.BlockSpec((dp1 - 1, TILE, hid), lambda i: (0, i, 0)),
            pl.BlockSpec((in_dim, hid), lambda i: (0, 0)),
            pl.BlockSpec((1, hid), lambda i: (0, 0)),
            pl.BlockSpec((dp1 - 1, in_dim, hid), lambda i: (0, 0, 0)),
            pl.BlockSpec((hid, fp_dim), lambda i: (0, 0)),
            pl.BlockSpec((NUM_GRAPHS, fp_dim), lambda i: (0, 0)),
        ],
        out_specs=[
            pl.BlockSpec((TILE, hid), lambda i: (i, 0)),
            pl.BlockSpec((NUM_GRAPHS, fp_dim), lambda i: (0, 0)),
        ],
        out_shape=[
            jax.ShapeDtypeStruct((n_pad, hid), jnp.float32),
            jax.ShapeDtypeStruct((NUM_GRAPHS, fp_dim), jnp.float32),
        ],
    )

    fp = jnp.zeros((NUM_GRAPHS, fp_dim), jnp.float32)
    xx = x_p
    for layer in range(num_layers):
        msgs = scatter_kernel(xx, packed_s, s2, e2, zero_f)
        xx, fp = tc_f(batch_p, xx, msgs, wst[layer], b2[layer],
                      wnt[layer], wfpt[layer], fp)
    return fp
